# Initial kernel scaffold; baseline (speedup 1.0000x reference)
import jax
import jax.numpy as jnp
from jax.experimental import pallas as pl


def kernel(x, edge_index, datanorm, W1, b1, W2, b2, W3, b3, Wc1, bc1, Wc2, bc2, We1, be1, We2, be2, We3, be3):
    raise NotImplementedError("write your pallas kernel here")


# speedup vs baseline: 4.3077x; 4.3077x over previous
import functools

import jax
import jax.numpy as jnp
from jax import lax
from jax.experimental import pallas as pl
from jax.experimental.pallas import tpu as pltpu
from jax.experimental.pallas import tpu_sc as plsc

N = 50000
E = 1600000
F = 32
NC = 2
NS = 16
NW = NC * NS
E_PER_W = E // NW
CB = 1000
NCHUNK = E_PER_W // CB
IDX_MINOR = 125
IDX_ROWS_PER_CHUNK = CB // IDX_MINOR
IDX_ROWS_TOTAL = E // IDX_MINOR
N_PER_TILE = N // NS

BN = 2000
BE = 8000


def _elu(v):
    return jnp.where(v > 0, v, jnp.exp(v) - 1.0)


def _node_body(x_ref, dn_ref, w1_ref, b1_ref, w2_ref, b2_ref, w3_ref, b3_ref,
               wa_ref, ba_ref, wb_ref, xa_ref, xb_ref):
    xn = x_ref[...] * dn_ref[...]
    h = jnp.tanh(jnp.dot(xn, w1_ref[...], preferred_element_type=jnp.float32) + b1_ref[...])
    h = jnp.tanh(jnp.dot(h, w2_ref[...], preferred_element_type=jnp.float32) + b2_ref[...])
    hh = jnp.tanh(jnp.dot(h, w3_ref[...], preferred_element_type=jnp.float32) + b3_ref[...])
    feat = jnp.concatenate([hh, xn, jnp.zeros((BN, 11), jnp.float32)], axis=1)
    xa_ref[...] = jnp.dot(feat, wa_ref[...], preferred_element_type=jnp.float32) + ba_ref[...]
    xb_ref[...] = jnp.dot(feat, wb_ref[...], preferred_element_type=jnp.float32)


def _node_stage(x, dn, W1, b1, W2, b2, W3, b3, Apad, bc1p, Bpad):
    full = lambda shape: pl.BlockSpec(shape, lambda i: (0, 0))
    return pl.pallas_call(
        _node_body,
        grid=(N // BN,),
        in_specs=[
            pl.BlockSpec((BN, 5), lambda i: (i, 0)),
            full((1, 5)), full((5, 32)), full((1, 32)), full((32, 32)),
            full((1, 32)), full((32, 16)), full((1, 16)),
            full((32, F)), full((1, F)), full((32, F)),
        ],
        out_specs=[pl.BlockSpec((BN, F), lambda i: (i, 0)),
                   pl.BlockSpec((BN, F), lambda i: (i, 0))],
        out_shape=[jax.ShapeDtypeStruct((N, F), jnp.float32),
                   jax.ShapeDtypeStruct((N, F), jnp.float32)],
    )(x, dn, W1, b1, W2, b2, W3, b3, Apad, bc1p, Bpad)


def _t1_body(z_ref, wc2_ref, bc2_ref, m_ref):
    p = _elu(z_ref[...])
    m_ref[...] = _elu(jnp.dot(p, wc2_ref[...], preferred_element_type=jnp.float32) + bc2_ref[...])


def _t1_stage(z1, Wc2p, bc2):
    return pl.pallas_call(
        _t1_body,
        grid=(E // BE,),
        in_specs=[pl.BlockSpec((BE, F), lambda i: (i, 0)),
                  pl.BlockSpec((F, 16), lambda i: (0, 0)),
                  pl.BlockSpec((1, 16), lambda i: (0, 0))],
        out_specs=pl.BlockSpec((BE, 16), lambda i: (i, 0)),
        out_shape=jax.ShapeDtypeStruct((E, 16), jnp.float32),
    )(z1, Wc2p, bc2)


def _combine_body(p_ref, wa_ref, ba_ref, wb_ref, ha_ref, hb_ref):
    h = p_ref[0] + p_ref[1]
    ha_ref[...] = jnp.dot(h, wa_ref[...], preferred_element_type=jnp.float32) + ba_ref[...]
    hb_ref[...] = jnp.dot(h, wb_ref[...], preferred_element_type=jnp.float32)


def _combine_stage(partial, Wa, be1, Wb):
    return pl.pallas_call(
        _combine_body,
        grid=(N // BN,),
        in_specs=[pl.BlockSpec((2, BN, 16), lambda i: (0, i, 0)),
                  pl.BlockSpec((16, F), lambda i: (0, 0)),
                  pl.BlockSpec((1, F), lambda i: (0, 0)),
                  pl.BlockSpec((16, F), lambda i: (0, 0))],
        out_specs=[pl.BlockSpec((BN, F), lambda i: (i, 0)),
                   pl.BlockSpec((BN, F), lambda i: (i, 0))],
        out_shape=[jax.ShapeDtypeStruct((N, F), jnp.float32),
                   jax.ShapeDtypeStruct((N, F), jnp.float32)],
    )(partial, Wa, be1, Wb)


def _t2_body(z_ref, w2_ref, b2_ref, w3_ref, b3_ref, o_ref):
    e = _elu(z_ref[...])
    e = _elu(jnp.dot(e, w2_ref[...], preferred_element_type=jnp.float32) + b2_ref[...])
    logits = jnp.dot(e, w3_ref[...], preferred_element_type=jnp.float32) + b3_ref[...]
    mx = jnp.max(logits, axis=-1, keepdims=True)
    lse = jnp.log(jnp.sum(jnp.exp(logits - mx), axis=-1, keepdims=True)) + mx
    o_ref[...] = logits - lse


def _t2_stage(z2, We2, be2, We3, be3):
    return pl.pallas_call(
        _t2_body,
        grid=(E // BE,),
        in_specs=[pl.BlockSpec((BE, F), lambda i: (i, 0)),
                  pl.BlockSpec((F, F), lambda i: (0, 0)),
                  pl.BlockSpec((1, F), lambda i: (0, 0)),
                  pl.BlockSpec((F, 4), lambda i: (0, 0)),
                  pl.BlockSpec((1, 4), lambda i: (0, 0))],
        out_specs=pl.BlockSpec((BE, 4), lambda i: (i, 0)),
        out_shape=jax.ShapeDtypeStruct((E, 4), jnp.float32),
    )(z2, We2, be2, We3, be3)


def _gather_add_body(ta_ref, tb_ref, ia_ref, ib_ref, out_ref,
                     ia_v, ib_v, ba_v, bb_v, sa, sb):
    wid = lax.axis_index("s") * NC + lax.axis_index("c")

    def chunk(ch, _):
        rbase = pl.multiple_of(wid * (NCHUNK * IDX_ROWS_PER_CHUNK) + ch * IDX_ROWS_PER_CHUNK, 8)
        ebase = pl.multiple_of(wid * E_PER_W + ch * CB, 8)
        pltpu.sync_copy(ia_ref.at[pl.ds(rbase, IDX_ROWS_PER_CHUNK)], ia_v)
        pltpu.sync_copy(ib_ref.at[pl.ds(rbase, IDX_ROWS_PER_CHUNK)], ib_v)
        cps = []
        for j in range(IDX_ROWS_PER_CHUNK):
            cps.append(pltpu.async_copy(
                ta_ref.at[ia_v.at[j]], ba_v.at[pl.ds(j * IDX_MINOR, IDX_MINOR)], sa))
            cps.append(pltpu.async_copy(
                tb_ref.at[ib_v.at[j]], bb_v.at[pl.ds(j * IDX_MINOR, IDX_MINOR)], sb))
        for cp in cps:
            cp.wait()

        def add_row(r, _):
            ba_v[r, pl.ds(0, 16)] = ba_v[r, pl.ds(0, 16)] + bb_v[r, pl.ds(0, 16)]
            ba_v[r, pl.ds(16, 16)] = ba_v[r, pl.ds(16, 16)] + bb_v[r, pl.ds(16, 16)]
            return 0

        lax.fori_loop(0, CB, add_row, 0)
        pltpu.sync_copy(ba_v, out_ref.at[pl.ds(ebase, CB)])
        return 0

    lax.fori_loop(0, NCHUNK, chunk, 0)


@functools.partial(
    pl.kernel,
    out_type=jax.ShapeDtypeStruct((E, F), jnp.float32),
    mesh=plsc.VectorSubcoreMesh(core_axis_name="c", subcore_axis_name="s"),
    compiler_params=pltpu.CompilerParams(use_tc_tiling_on_sc=False),
    scratch_types=[
        pltpu.VMEM((IDX_ROWS_PER_CHUNK, IDX_MINOR), jnp.int32),
        pltpu.VMEM((IDX_ROWS_PER_CHUNK, IDX_MINOR), jnp.int32),
        pltpu.VMEM((CB, F), jnp.float32),
        pltpu.VMEM((CB, F), jnp.float32),
        pltpu.SemaphoreType.DMA,
        pltpu.SemaphoreType.DMA,
    ],
)
def _gather_add(ta_ref, tb_ref, ia_ref, ib_ref, out_ref, ia_v, ib_v, ba_v, bb_v, sa, sb):
    _gather_add_body(ta_ref, tb_ref, ia_ref, ib_ref, out_ref,
                     ia_v, ib_v, ba_v, bb_v, sa, sb)


@functools.partial(
    pl.kernel,
    out_type=jax.ShapeDtypeStruct((NC, N, 16), jnp.float32),
    mesh=plsc.VectorSubcoreMesh(core_axis_name="c", subcore_axis_name="s"),
    compiler_params=pltpu.CompilerParams(use_tc_tiling_on_sc=False),
    scratch_types=[
        pltpu.VMEM((IDX_ROWS_PER_CHUNK, IDX_MINOR), jnp.int32),
        pltpu.VMEM((CB, 16), jnp.float32),
        pltpu.VMEM((N_PER_TILE, 16), jnp.float32),
        pltpu.VMEM_SHARED((N, 16), jnp.float32),
    ],
)
def _scatter_add(m_ref, col_ref, out_ref, idx_v, m_v, stage_v, acc_sh):
    c = lax.axis_index("c")
    s = lax.axis_index("s")
    wid = s * NC + c

    def zrow(r, _):
        stage_v[r, :] = jnp.zeros((16,), jnp.float32)
        return 0

    lax.fori_loop(0, N_PER_TILE, zrow, 0)
    pltpu.sync_copy(stage_v, acc_sh.at[pl.ds(s * N_PER_TILE, N_PER_TILE)])
    plsc.subcore_barrier()

    def chunk(ch, _):
        rbase = pl.multiple_of(wid * (NCHUNK * IDX_ROWS_PER_CHUNK) + ch * IDX_ROWS_PER_CHUNK, 8)
        ebase = pl.multiple_of(wid * E_PER_W + ch * CB, 8)
        pltpu.sync_copy(col_ref.at[pl.ds(rbase, IDX_ROWS_PER_CHUNK)], idx_v)
        pltpu.sync_copy(m_ref.at[pl.ds(ebase, CB)], m_v)
        for j in range(IDX_ROWS_PER_CHUNK):
            pltpu.sync_copy(m_v.at[pl.ds(j * IDX_MINOR, IDX_MINOR)],
                            acc_sh.at[idx_v.at[j]], add=True)
        return 0

    lax.fori_loop(0, NCHUNK, chunk, 0)
    plsc.subcore_barrier()
    pltpu.sync_copy(acc_sh.at[pl.ds(s * N_PER_TILE, N_PER_TILE)], stage_v)
    pltpu.sync_copy(stage_v, out_ref.at[c, pl.ds(s * N_PER_TILE, N_PER_TILE)])


def kernel(x, edge_index, datanorm, W1, b1, W2, b2, W3, b3,
           Wc1, bc1, Wc2, bc2, We1, be1, We2, be2, We3, be3):
    row = edge_index[0].reshape(IDX_ROWS_TOTAL, IDX_MINOR)
    col = edge_index[1].reshape(IDX_ROWS_TOTAL, IDX_MINOR)

    A = Wc1[:21] - Wc1[21:]
    Bm = Wc1[21:]
    Apad = jnp.zeros((32, F), jnp.float32).at[:21, :29].set(A)
    Bpad = jnp.zeros((32, F), jnp.float32).at[:21, :29].set(Bm)
    bc1p = jnp.zeros((1, F), jnp.float32).at[0, :29].set(bc1)
    Wc2p = jnp.zeros((F, 16), jnp.float32).at[:29].set(Wc2)
    We1a = We1[:16]
    We1b = We1[16:]

    XA, XB = _node_stage(x, datanorm.reshape(1, 5), W1, b1.reshape(1, 32),
                         W2, b2.reshape(1, 32), W3, b3.reshape(1, 16),
                         Apad, bc1p, Bpad)
    z1 = _gather_add(XA, XB, col, row)
    m = _t1_stage(z1, Wc2p, bc2.reshape(1, 16))
    partial = _scatter_add(m, col)
    HA, HB = _combine_stage(partial, We1a, be1.reshape(1, F), We1b)
    z2 = _gather_add(HA, HB, row, col)
    return _t2_stage(z2, We2, be2.reshape(1, F), We3, be3.reshape(1, 4))


# speedup vs baseline: 5.1172x; 1.1879x over previous
import functools

import jax
import jax.numpy as jnp
from jax import lax
from jax.experimental import pallas as pl
from jax.experimental.pallas import tpu as pltpu
from jax.experimental.pallas import tpu_sc as plsc

N = 50000
E = 1600000
F = 32
NC = 2
NS = 16
NW = NC * NS
E_PER_W = E // NW
CB = 1000
NCHUNK = E_PER_W // CB
NCH = E // CB
PR = CB // 4
IB = 125
N_PER_TILE = N // NS

BN = 2000
BR = 2000


def _elu(v):
    return jnp.where(v > 0, v, jnp.exp(v) - 1.0)


def _node_body(x_ref, dn_ref, w1_ref, b1_ref, w2_ref, b2_ref, w3_ref, b3_ref,
               wa_ref, ba_ref, wb_ref, xa_ref, xb_ref):
    xn = x_ref[...] * dn_ref[...]
    h = jnp.tanh(jnp.dot(xn, w1_ref[...], preferred_element_type=jnp.float32) + b1_ref[...])
    h = jnp.tanh(jnp.dot(h, w2_ref[...], preferred_element_type=jnp.float32) + b2_ref[...])
    hh = jnp.tanh(jnp.dot(h, w3_ref[...], preferred_element_type=jnp.float32) + b3_ref[...])
    feat = jnp.concatenate([hh, xn, jnp.zeros((BN, 11), jnp.float32)], axis=1)
    xa_ref[...] = jnp.dot(feat, wa_ref[...], preferred_element_type=jnp.float32) + ba_ref[...]
    xb_ref[...] = jnp.dot(feat, wb_ref[...], preferred_element_type=jnp.float32)


def _node_stage(x, dn, W1, b1, W2, b2, W3, b3, Apad, bc1p, Bpad):
    full = lambda shape: pl.BlockSpec(shape, lambda i: (0, 0))
    return pl.pallas_call(
        _node_body,
        grid=(N // BN,),
        in_specs=[
            pl.BlockSpec((BN, 5), lambda i: (i, 0)),
            full((1, 5)), full((5, 32)), full((1, 32)), full((32, 32)),
            full((1, 32)), full((32, 16)), full((1, 16)),
            full((32, F)), full((1, F)), full((32, F)),
        ],
        out_specs=[pl.BlockSpec((BN, F), lambda i: (i, 0)),
                   pl.BlockSpec((BN, F), lambda i: (i, 0))],
        out_shape=[jax.ShapeDtypeStruct((N, F), jnp.float32),
                   jax.ShapeDtypeStruct((N, F), jnp.float32)],
    )(x, dn, W1, b1, W2, b2, W3, b3, Apad, bc1p, Bpad)


def _t1_body(z_ref, wbd_ref, bbd_ref, m_ref):
    p = _elu(z_ref[...])
    mm = _elu(jnp.dot(p, wbd_ref[...], preferred_element_type=jnp.float32) + bbd_ref[...])
    m_ref[...] = jnp.concatenate([mm, jnp.zeros((BR, 64), jnp.float32)], axis=1)


def _t1_stage(z1, Wc2bd, bc2bd):
    return pl.pallas_call(
        _t1_body,
        grid=(E // 4 // BR,),
        in_specs=[pl.BlockSpec((BR, 128), lambda i: (i, 0)),
                  pl.BlockSpec((128, 64), lambda i: (0, 0)),
                  pl.BlockSpec((1, 64), lambda i: (0, 0))],
        out_specs=pl.BlockSpec((BR, 128), lambda i: (i, 0)),
        out_shape=jax.ShapeDtypeStruct((E // 4, 128), jnp.float32),
    )(z1, Wc2bd, bc2bd)


def _combine_body(p_ref, wa_ref, ba_ref, wb_ref, ha_ref, hb_ref):
    h = p_ref[0] + p_ref[1]
    ha_ref[...] = jnp.dot(h, wa_ref[...], preferred_element_type=jnp.float32) + ba_ref[...]
    hb_ref[...] = jnp.dot(h, wb_ref[...], preferred_element_type=jnp.float32)


def _combine_stage(partial, Wa, be1, Wb):
    return pl.pallas_call(
        _combine_body,
        grid=(N // BN,),
        in_specs=[pl.BlockSpec((2, BN, 16), lambda i: (0, i, 0)),
                  pl.BlockSpec((16, F), lambda i: (0, 0)),
                  pl.BlockSpec((1, F), lambda i: (0, 0)),
                  pl.BlockSpec((16, F), lambda i: (0, 0))],
        out_specs=[pl.BlockSpec((BN, F), lambda i: (i, 0)),
                   pl.BlockSpec((BN, F), lambda i: (i, 0))],
        out_shape=[jax.ShapeDtypeStruct((N, F), jnp.float32),
                   jax.ShapeDtypeStruct((N, F), jnp.float32)],
    )(partial, Wa, be1, Wb)


def _t2_body(z_ref, w2_ref, b2_ref, w3_ref, b3_ref, p1_ref, p2_ref, p3_ref,
             s_ref, g_ref, o_ref):
    e = _elu(z_ref[...])
    e = _elu(jnp.dot(e, w2_ref[...], preferred_element_type=jnp.float32) + b2_ref[...])
    l = jnp.dot(e, w3_ref[...], preferred_element_type=jnp.float32) + b3_ref[...]
    l1 = jnp.dot(l, p1_ref[...], preferred_element_type=jnp.float32)
    l2 = jnp.dot(l, p2_ref[...], preferred_element_type=jnp.float32)
    l3 = jnp.dot(l, p3_ref[...], preferred_element_type=jnp.float32)
    mx = jnp.maximum(jnp.maximum(l, l1), jnp.maximum(l2, l3))
    bmx = jnp.dot(mx, s_ref[...], preferred_element_type=jnp.float32)
    sh = l - bmx
    ssum = jnp.dot(jnp.exp(sh), g_ref[...], preferred_element_type=jnp.float32)
    o_ref[...] = sh - jnp.log(ssum)


def _t2_stage(z2, We2bd, be2bd, We3bd, be3bd, P1, P2, P3, S16, G16):
    full = lambda shape: pl.BlockSpec(shape, lambda i: (0, 0))
    return pl.pallas_call(
        _t2_body,
        grid=(E // 4 // BR,),
        in_specs=[pl.BlockSpec((BR, 128), lambda i: (i, 0)),
                  full((128, 128)), full((1, 128)), full((128, 16)),
                  full((1, 16)), full((16, 16)), full((16, 16)),
                  full((16, 16)), full((16, 16)), full((16, 16))],
        out_specs=pl.BlockSpec((BR, 16), lambda i: (i, 0)),
        out_shape=jax.ShapeDtypeStruct((E // 4, 16), jnp.float32),
    )(z2, We2bd, be2bd, We3bd, be3bd, P1, P2, P3, S16, G16)


@functools.partial(
    pl.kernel,
    out_type=jax.ShapeDtypeStruct((NCH, PR * 128), jnp.float32),
    mesh=plsc.VectorSubcoreMesh(core_axis_name="c", subcore_axis_name="s"),
    compiler_params=pltpu.CompilerParams(use_tc_tiling_on_sc=False),
    scratch_types=[
        pltpu.VMEM((8, IB), jnp.int32),
        pltpu.VMEM((8, IB), jnp.int32),
        pltpu.VMEM((CB, F), jnp.float32),
        pltpu.VMEM((CB, F), jnp.float32),
        pltpu.VMEM((PR * 128,), jnp.float32),
        pltpu.SemaphoreType.DMA,
        pltpu.SemaphoreType.DMA,
    ],
)
def _gather_add(ta_ref, tb_ref, ia_ref, ib_ref, out_ref, ia_v, ib_v, ba_v, bb_v, bp_v, sa, sb):
    wid = lax.axis_index("s") * NC + lax.axis_index("c")

    def chunk(ch, _):
        gch = wid * NCHUNK + ch
        rbase = pl.multiple_of(8 * gch, 8)
        pltpu.sync_copy(ia_ref.at[pl.ds(rbase, 8)], ia_v)
        pltpu.sync_copy(ib_ref.at[pl.ds(rbase, 8)], ib_v)
        cps = []
        for b in range(8):
            dst = pl.ds(IB * b, IB)
            cps.append(pltpu.async_copy(ta_ref.at[ia_v.at[b]], ba_v.at[dst], sa))
            cps.append(pltpu.async_copy(tb_ref.at[ib_v.at[b]], bb_v.at[dst], sb))
        for cp in cps:
            cp.wait()

        def add_row(r, _):
            for j in range(4):
                p = PR * j + r
                for k in range(2):
                    bp_v[pl.ds(128 * r + F * j + 16 * k, 16)] = (
                        ba_v[p, pl.ds(16 * k, 16)] + bb_v[p, pl.ds(16 * k, 16)])
            return 0

        lax.fori_loop(0, PR, add_row, 0)
        pltpu.sync_copy(bp_v, out_ref.at[gch])
        return 0

    lax.fori_loop(0, NCHUNK, chunk, 0)


@functools.partial(
    pl.kernel,
    out_type=jax.ShapeDtypeStruct((NC, N, 16), jnp.float32),
    mesh=plsc.VectorSubcoreMesh(core_axis_name="c", subcore_axis_name="s"),
    compiler_params=pltpu.CompilerParams(use_tc_tiling_on_sc=False),
    scratch_types=[
        pltpu.VMEM((8, IB), jnp.int32),
        pltpu.VMEM((PR * 128,), jnp.float32),
        pltpu.VMEM((CB, 16), jnp.float32),
        pltpu.VMEM_SHARED((N, 16), jnp.float32),
    ],
)
def _scatter_add(m_ref, col_ref, out_ref, idx_v, m_v, mc_v, acc_sh):
    c = lax.axis_index("c")
    s = lax.axis_index("s")
    wid = s * NC + c

    def zrow(r, _):
        mc_v[r, :] = jnp.zeros((16,), jnp.float32)
        return 0

    lax.fori_loop(0, CB, zrow, 0)
    for k in range(3):
        pltpu.sync_copy(mc_v, acc_sh.at[pl.ds(s * N_PER_TILE + k * CB, CB)])
    pltpu.sync_copy(mc_v.at[pl.ds(0, IB)],
                    acc_sh.at[pl.ds(s * N_PER_TILE + 3 * CB, IB)])
    plsc.subcore_barrier()

    def chunk(ch, _):
        gch = wid * NCHUNK + ch
        rbase = pl.multiple_of(8 * gch, 8)
        pltpu.sync_copy(col_ref.at[pl.ds(rbase, 8)], idx_v)
        pltpu.sync_copy(m_ref.at[gch], m_v)

        def unpack_row(r, _):
            for j in range(4):
                mc_v[4 * r + j, :] = m_v[pl.ds(128 * r + 16 * j, 16)]
            return 0

        lax.fori_loop(0, PR, unpack_row, 0)
        for b in range(8):
            pltpu.sync_copy(mc_v.at[pl.ds(IB * b, IB)],
                            acc_sh.at[idx_v.at[b]], add=True)
        return 0

    lax.fori_loop(0, NCHUNK, chunk, 0)
    plsc.subcore_barrier()
    for k in range(3):
        pltpu.sync_copy(acc_sh.at[pl.ds(s * N_PER_TILE + k * CB, CB)], mc_v)
        pltpu.sync_copy(mc_v, out_ref.at[c, pl.ds(s * N_PER_TILE + k * CB, CB)])
    pltpu.sync_copy(acc_sh.at[pl.ds(s * N_PER_TILE + 3 * CB, IB)],
                    mc_v.at[pl.ds(0, IB)])
    pltpu.sync_copy(mc_v.at[pl.ds(0, IB)],
                    out_ref.at[c, pl.ds(s * N_PER_TILE + 3 * CB, IB)])


def _chunked_idx(v):
    return v.reshape(NCH, 2, IB, 4).transpose(0, 3, 1, 2).reshape(NCH * 8, IB)


def kernel(x, edge_index, datanorm, W1, b1, W2, b2, W3, b3,
           Wc1, bc1, Wc2, bc2, We1, be1, We2, be2, We3, be3):
    row3 = _chunked_idx(edge_index[0])
    col3 = _chunked_idx(edge_index[1])
    col3p = edge_index[1].reshape(NCH * 8, IB)

    A = Wc1[:21] - Wc1[21:]
    Bm = Wc1[21:]
    Apad = jnp.zeros((32, F), jnp.float32).at[:21, :29].set(A)
    Bpad = jnp.zeros((32, F), jnp.float32).at[:21, :29].set(Bm)
    bc1p = jnp.zeros((1, F), jnp.float32).at[0, :29].set(bc1)
    Wc2p = jnp.zeros((F, 16), jnp.float32).at[:29].set(Wc2)

    eye4 = jnp.eye(4, dtype=jnp.float32)
    Wc2bd = jnp.kron(eye4, Wc2p)
    bc2bd = jnp.tile(bc2, 4).reshape(1, 64)
    We2bd = jnp.kron(eye4, We2)
    be2bd = jnp.tile(be2, 4).reshape(1, 128)
    We3bd = jnp.kron(eye4, We3)
    be3bd = jnp.tile(be3, 4).reshape(1, 16)
    P1 = jnp.eye(16, k=-1, dtype=jnp.float32)
    P2 = jnp.eye(16, k=-2, dtype=jnp.float32)
    P3 = jnp.eye(16, k=-3, dtype=jnp.float32)
    spread = jnp.zeros((4, 4), jnp.float32).at[0].set(1.0)
    S16 = jnp.kron(eye4, spread)
    G16 = jnp.kron(eye4, jnp.ones((4, 4), jnp.float32))

    XA, XB = _node_stage(x, datanorm.reshape(1, 5), W1, b1.reshape(1, 32),
                         W2, b2.reshape(1, 32), W3, b3.reshape(1, 16),
                         Apad, bc1p, Bpad)
    z1 = _gather_add(XA, XB, col3, row3).reshape(E // 4, 128)
    m = _t1_stage(z1, Wc2bd, bc2bd).reshape(NCH, PR * 128)
    partial = _scatter_add(m, col3p)
    HA, HB = _combine_stage(partial, We1[:16], be1.reshape(1, F), We1[16:])
    z2 = _gather_add(HA, HB, row3, col3).reshape(E // 4, 128)
    out = _t2_stage(z2, We2bd, be2bd, We3bd, be3bd, P1, P2, P3, S16, G16)
    return out.reshape(E, 4)


# speedup vs baseline: 5.7220x; 1.1182x over previous
import functools

import jax
import jax.numpy as jnp
from jax import lax
from jax.experimental import pallas as pl
from jax.experimental.pallas import tpu as pltpu
from jax.experimental.pallas import tpu_sc as plsc

N = 50000
E = 1600000
F = 32
NC = 2
NS = 16
NW = NC * NS
E_PER_W = E // NW
CB = 1000
NCHUNK = E_PER_W // CB
NCH = E // CB
PR = CB // 4
IB = 125
N_PER_TILE = N // NS

BN = 2000
BR = 2000


def _elu(v):
    return jnp.where(v > 0, v, jnp.exp(v) - 1.0)


def _node_body(x_ref, dn_ref, w1_ref, b1_ref, w2_ref, b2_ref, w3_ref, b3_ref,
               wa_ref, ba_ref, wb_ref, xa_ref, xb_ref):
    xn = x_ref[...] * dn_ref[...]
    h = jnp.tanh(jnp.dot(xn, w1_ref[...], preferred_element_type=jnp.float32) + b1_ref[...])
    h = jnp.tanh(jnp.dot(h, w2_ref[...], preferred_element_type=jnp.float32) + b2_ref[...])
    hh = jnp.tanh(jnp.dot(h, w3_ref[...], preferred_element_type=jnp.float32) + b3_ref[...])
    feat = jnp.concatenate([hh, xn, jnp.zeros((BN, 11), jnp.float32)], axis=1)
    xa_ref[...] = jnp.dot(feat, wa_ref[...], preferred_element_type=jnp.float32) + ba_ref[...]
    xb_ref[...] = jnp.dot(feat, wb_ref[...], preferred_element_type=jnp.float32)


def _node_stage(x, dn, W1, b1, W2, b2, W3, b3, Apad, bc1p, Bpad):
    full = lambda shape: pl.BlockSpec(shape, lambda i: (0, 0))
    return pl.pallas_call(
        _node_body,
        grid=(N // BN,),
        in_specs=[
            pl.BlockSpec((BN, 5), lambda i: (i, 0)),
            full((1, 5)), full((5, 32)), full((1, 32)), full((32, 32)),
            full((1, 32)), full((32, 16)), full((1, 16)),
            full((32, F)), full((1, F)), full((32, F)),
        ],
        out_specs=[pl.BlockSpec((BN, F), lambda i: (i, 0)),
                   pl.BlockSpec((BN, F), lambda i: (i, 0))],
        out_shape=[jax.ShapeDtypeStruct((N, F), jnp.float32),
                   jax.ShapeDtypeStruct((N, F), jnp.float32)],
    )(x, dn, W1, b1, W2, b2, W3, b3, Apad, bc1p, Bpad)


def _t1_body(z_ref, wbd_ref, bbd_ref, m_ref):
    p = _elu(z_ref[...])
    mm = _elu(jnp.dot(p, wbd_ref[...], preferred_element_type=jnp.float32) + bbd_ref[...])
    m_ref[...] = jnp.concatenate([mm, jnp.zeros((BR, 64), jnp.float32)], axis=1)


def _t1_stage(z1, Wc2bd, bc2bd):
    return pl.pallas_call(
        _t1_body,
        grid=(E // 4 // BR,),
        in_specs=[pl.BlockSpec((BR, 128), lambda i: (i, 0)),
                  pl.BlockSpec((128, 64), lambda i: (0, 0)),
                  pl.BlockSpec((1, 64), lambda i: (0, 0))],
        out_specs=pl.BlockSpec((BR, 128), lambda i: (i, 0)),
        out_shape=jax.ShapeDtypeStruct((E // 4, 128), jnp.float32),
    )(z1, Wc2bd, bc2bd)


def _combine_body(p_ref, wa_ref, ba_ref, wb_ref, ha_ref, hb_ref):
    h = p_ref[0] + p_ref[1]
    ha_ref[...] = jnp.dot(h, wa_ref[...], preferred_element_type=jnp.float32) + ba_ref[...]
    hb_ref[...] = jnp.dot(h, wb_ref[...], preferred_element_type=jnp.float32)


def _combine_stage(partial, Wa, be1, Wb):
    return pl.pallas_call(
        _combine_body,
        grid=(N // BN,),
        in_specs=[pl.BlockSpec((2, BN, 16), lambda i: (0, i, 0)),
                  pl.BlockSpec((16, F), lambda i: (0, 0)),
                  pl.BlockSpec((1, F), lambda i: (0, 0)),
                  pl.BlockSpec((16, F), lambda i: (0, 0))],
        out_specs=[pl.BlockSpec((BN, F), lambda i: (i, 0)),
                   pl.BlockSpec((BN, F), lambda i: (i, 0))],
        out_shape=[jax.ShapeDtypeStruct((N, F), jnp.float32),
                   jax.ShapeDtypeStruct((N, F), jnp.float32)],
    )(partial, Wa, be1, Wb)


def _t2_body(z_ref, w2_ref, b2_ref, w3_ref, b3_ref, p1_ref, p2_ref, p3_ref,
             s_ref, g_ref, o_ref):
    e = _elu(z_ref[...])
    e = _elu(jnp.dot(e, w2_ref[...], preferred_element_type=jnp.float32) + b2_ref[...])
    l = jnp.dot(e, w3_ref[...], preferred_element_type=jnp.float32) + b3_ref[...]
    l1 = jnp.dot(l, p1_ref[...], preferred_element_type=jnp.float32)
    l2 = jnp.dot(l, p2_ref[...], preferred_element_type=jnp.float32)
    l3 = jnp.dot(l, p3_ref[...], preferred_element_type=jnp.float32)
    mx = jnp.maximum(jnp.maximum(l, l1), jnp.maximum(l2, l3))
    bmx = jnp.dot(mx, s_ref[...], preferred_element_type=jnp.float32)
    sh = l - bmx
    ssum = jnp.dot(jnp.exp(sh), g_ref[...], preferred_element_type=jnp.float32)
    o_ref[...] = jnp.concatenate(
        [sh - jnp.log(ssum), jnp.zeros((BR, 112), jnp.float32)], axis=1)


def _t2_stage(z2, We2bd, be2bd, We3bd, be3bd, P1, P2, P3, S16, G16):
    full = lambda shape: pl.BlockSpec(shape, lambda i: (0, 0))
    return pl.pallas_call(
        _t2_body,
        grid=(E // 4 // BR,),
        in_specs=[pl.BlockSpec((BR, 128), lambda i: (i, 0)),
                  full((128, 128)), full((1, 128)), full((128, 16)),
                  full((1, 16)), full((16, 16)), full((16, 16)),
                  full((16, 16)), full((16, 16)), full((16, 16))],
        out_specs=pl.BlockSpec((BR, 128), lambda i: (i, 0)),
        out_shape=jax.ShapeDtypeStruct((E // 4, 128), jnp.float32),
    )(z2, We2bd, be2bd, We3bd, be3bd, P1, P2, P3, S16, G16)


@functools.partial(
    pl.kernel,
    out_type=jax.ShapeDtypeStruct((NCH, CB, F), jnp.float32),
    mesh=plsc.VectorSubcoreMesh(core_axis_name="c", subcore_axis_name="s"),
    compiler_params=pltpu.CompilerParams(use_tc_tiling_on_sc=False),
    scratch_types=[
        pltpu.VMEM((8, IB), jnp.int32),
        pltpu.VMEM((8, IB), jnp.int32),
        pltpu.VMEM((CB, F), jnp.float32),
        pltpu.VMEM((CB, F), jnp.float32),
        pltpu.SemaphoreType.DMA,
        pltpu.SemaphoreType.DMA,
    ],
)
def _gather_add(ta_ref, tb_ref, ia_ref, ib_ref, out_ref, ia_v, ib_v, ba_v, bb_v, sa, sb):
    wid = lax.axis_index("s") * NC + lax.axis_index("c")

    def chunk(ch, _):
        gch = wid * NCHUNK + ch
        rbase = pl.multiple_of(8 * gch, 8)
        pltpu.sync_copy(ia_ref.at[pl.ds(rbase, 8)], ia_v)
        pltpu.sync_copy(ib_ref.at[pl.ds(rbase, 8)], ib_v)
        cps = []
        for b in range(8):
            dst = pl.ds(IB * b, IB)
            cps.append(pltpu.async_copy(ta_ref.at[ia_v.at[b]], ba_v.at[dst], sa))
            cps.append(pltpu.async_copy(tb_ref.at[ib_v.at[b]], bb_v.at[dst], sb))
        for cp in cps:
            cp.wait()

        def add_row(r, _):
            for k in range(2):
                ba_v[r, pl.ds(16 * k, 16)] = (
                    ba_v[r, pl.ds(16 * k, 16)] + bb_v[r, pl.ds(16 * k, 16)])
            return 0

        lax.fori_loop(0, CB, add_row, 0)
        pltpu.sync_copy(ba_v, out_ref.at[gch])
        return 0

    lax.fori_loop(0, NCHUNK, chunk, 0)


@functools.partial(
    pl.kernel,
    out_type=jax.ShapeDtypeStruct((NC, N, 16), jnp.float32),
    mesh=plsc.VectorSubcoreMesh(core_axis_name="c", subcore_axis_name="s"),
    compiler_params=pltpu.CompilerParams(use_tc_tiling_on_sc=False),
    scratch_types=[
        pltpu.VMEM((8, IB), jnp.int32),
        pltpu.VMEM((PR * 128,), jnp.float32),
        pltpu.VMEM((CB, 16), jnp.float32),
        pltpu.VMEM_SHARED((N, 16), jnp.float32),
    ],
)
def _scatter_add(m_ref, col_ref, out_ref, idx_v, m_v, mc_v, acc_sh):
    c = lax.axis_index("c")
    s = lax.axis_index("s")
    wid = s * NC + c

    def zrow(r, _):
        mc_v[r, :] = jnp.zeros((16,), jnp.float32)
        return 0

    lax.fori_loop(0, CB, zrow, 0)
    for k in range(3):
        pltpu.sync_copy(mc_v, acc_sh.at[pl.ds(s * N_PER_TILE + k * CB, CB)])
    pltpu.sync_copy(mc_v.at[pl.ds(0, IB)],
                    acc_sh.at[pl.ds(s * N_PER_TILE + 3 * CB, IB)])
    plsc.subcore_barrier()

    def chunk(ch, _):
        gch = wid * NCHUNK + ch
        rbase = pl.multiple_of(8 * gch, 8)
        pltpu.sync_copy(col_ref.at[pl.ds(rbase, 8)], idx_v)
        pltpu.sync_copy(m_ref.at[gch], m_v)

        def unpack_row(r, _):
            for j in range(4):
                mc_v[4 * r + j, :] = m_v[pl.ds(128 * r + 16 * j, 16)]
            return 0

        lax.fori_loop(0, PR, unpack_row, 0)
        for b in range(8):
            pltpu.sync_copy(mc_v.at[pl.ds(IB * b, IB)],
                            acc_sh.at[idx_v.at[b]], add=True)
        return 0

    lax.fori_loop(0, NCHUNK, chunk, 0)
    plsc.subcore_barrier()
    for k in range(3):
        pltpu.sync_copy(acc_sh.at[pl.ds(s * N_PER_TILE + k * CB, CB)], mc_v)
        pltpu.sync_copy(mc_v, out_ref.at[c, pl.ds(s * N_PER_TILE + k * CB, CB)])
    pltpu.sync_copy(acc_sh.at[pl.ds(s * N_PER_TILE + 3 * CB, IB)],
                    mc_v.at[pl.ds(0, IB)])
    pltpu.sync_copy(mc_v.at[pl.ds(0, IB)],
                    out_ref.at[c, pl.ds(s * N_PER_TILE + 3 * CB, IB)])


@functools.partial(
    pl.kernel,
    out_type=jax.ShapeDtypeStruct((NCH, CB * 4), jnp.float32),
    mesh=plsc.VectorSubcoreMesh(core_axis_name="c", subcore_axis_name="s"),
    compiler_params=pltpu.CompilerParams(use_tc_tiling_on_sc=False),
    scratch_types=[
        pltpu.VMEM((PR * 128,), jnp.float32),
        pltpu.VMEM((CB * 4,), jnp.float32),
    ],
)
def _fmt_out(lg_ref, out_ref, m_v, ov_v):
    wid = lax.axis_index("s") * NC + lax.axis_index("c")

    def chunk(ch, _):
        gch = wid * NCHUNK + ch
        pltpu.sync_copy(lg_ref.at[gch], m_v)

        def row(r, _):
            ov_v[pl.ds(16 * r, 16)] = m_v[pl.ds(128 * r, 16)]
            return 0

        lax.fori_loop(0, PR, row, 0)
        pltpu.sync_copy(ov_v, out_ref.at[gch])
        return 0

    lax.fori_loop(0, NCHUNK, chunk, 0)


def kernel(x, edge_index, datanorm, W1, b1, W2, b2, W3, b3,
           Wc1, bc1, Wc2, bc2, We1, be1, We2, be2, We3, be3):
    row3 = edge_index[0].reshape(NCH * 8, IB)
    col3 = edge_index[1].reshape(NCH * 8, IB)
    col3p = col3

    A = Wc1[:21] - Wc1[21:]
    Bm = Wc1[21:]
    Apad = jnp.zeros((32, F), jnp.float32).at[:21, :29].set(A)
    Bpad = jnp.zeros((32, F), jnp.float32).at[:21, :29].set(Bm)
    bc1p = jnp.zeros((1, F), jnp.float32).at[0, :29].set(bc1)
    Wc2p = jnp.zeros((F, 16), jnp.float32).at[:29].set(Wc2)

    eye4 = jnp.eye(4, dtype=jnp.float32)
    Wc2bd = jnp.kron(eye4, Wc2p)
    bc2bd = jnp.tile(bc2, 4).reshape(1, 64)
    We2bd = jnp.kron(eye4, We2)
    be2bd = jnp.tile(be2, 4).reshape(1, 128)
    We3bd = jnp.kron(eye4, We3)
    be3bd = jnp.tile(be3, 4).reshape(1, 16)
    P1 = jnp.eye(16, k=-1, dtype=jnp.float32)
    P2 = jnp.eye(16, k=-2, dtype=jnp.float32)
    P3 = jnp.eye(16, k=-3, dtype=jnp.float32)
    spread = jnp.zeros((4, 4), jnp.float32).at[0].set(1.0)
    S16 = jnp.kron(eye4, spread)
    G16 = jnp.kron(eye4, jnp.ones((4, 4), jnp.float32))

    XA, XB = _node_stage(x, datanorm.reshape(1, 5), W1, b1.reshape(1, 32),
                         W2, b2.reshape(1, 32), W3, b3.reshape(1, 16),
                         Apad, bc1p, Bpad)
    z1 = _gather_add(XA, XB, col3, row3).reshape(E // 4, 128)
    m = _t1_stage(z1, Wc2bd, bc2bd).reshape(NCH, PR * 128)
    partial = _scatter_add(m, col3p)
    HA, HB = _combine_stage(partial, We1[:16], be1.reshape(1, F), We1[16:])
    z2 = _gather_add(HA, HB, row3, col3).reshape(E // 4, 128)
    out = _t2_stage(z2, We2bd, be2bd, We3bd, be3bd, P1, P2, P3, S16, G16)
    return _fmt_out(out.reshape(NCH, PR * 128)).reshape(E, 4)


# speedup vs baseline: 6.0833x; 1.0631x over previous
import functools

import jax
import jax.numpy as jnp
from jax import lax
from jax.experimental import pallas as pl
from jax.experimental.pallas import tpu as pltpu
from jax.experimental.pallas import tpu_sc as plsc

N = 50000
E = 1600000
F = 32
NC = 2
NS = 16
NW = NC * NS
E_PER_W = E // NW
CB = 1000
NCHUNK = E_PER_W // CB
NCH = E // CB
PR = CB // 4
IB = 125
N_PER_TILE = N // NS

BN = 2000
BR = 2000


def _elu(v):
    return jnp.where(v > 0, v, jnp.exp(v) - 1.0)


def _node_body(x_ref, dn_ref, w1_ref, b1_ref, w2_ref, b2_ref, w3_ref, b3_ref,
               wa_ref, ba_ref, wb_ref, xa_ref, xb_ref):
    xn = x_ref[...] * dn_ref[...]
    h = jnp.tanh(jnp.dot(xn, w1_ref[...], preferred_element_type=jnp.float32) + b1_ref[...])
    h = jnp.tanh(jnp.dot(h, w2_ref[...], preferred_element_type=jnp.float32) + b2_ref[...])
    hh = jnp.tanh(jnp.dot(h, w3_ref[...], preferred_element_type=jnp.float32) + b3_ref[...])
    feat = jnp.concatenate([hh, xn, jnp.zeros((BN, 11), jnp.float32)], axis=1)
    xa_ref[...] = jnp.dot(feat, wa_ref[...], preferred_element_type=jnp.float32) + ba_ref[...]
    xb_ref[...] = jnp.dot(feat, wb_ref[...], preferred_element_type=jnp.float32)


def _node_stage(x, dn, W1, b1, W2, b2, W3, b3, Apad, bc1p, Bpad):
    full = lambda shape: pl.BlockSpec(shape, lambda i: (0, 0))
    return pl.pallas_call(
        _node_body,
        grid=(N // BN,),
        in_specs=[
            pl.BlockSpec((BN, 5), lambda i: (i, 0)),
            full((1, 5)), full((5, 32)), full((1, 32)), full((32, 32)),
            full((1, 32)), full((32, 16)), full((1, 16)),
            full((32, F)), full((1, F)), full((32, F)),
        ],
        out_specs=[pl.BlockSpec((BN, F), lambda i: (i, 0)),
                   pl.BlockSpec((BN, F), lambda i: (i, 0))],
        out_shape=[jax.ShapeDtypeStruct((N, F), jnp.float32),
                   jax.ShapeDtypeStruct((N, F), jnp.float32)],
    )(x, dn, W1, b1, W2, b2, W3, b3, Apad, bc1p, Bpad)


def _t1_body(z_ref, wbd_ref, bbd_ref, m_ref):
    p = _elu(z_ref[...])
    mm = _elu(jnp.dot(p, wbd_ref[...], preferred_element_type=jnp.float32) + bbd_ref[...])
    m_ref[...] = jnp.concatenate([mm, jnp.zeros((BR, 64), jnp.float32)], axis=1)


def _t1_stage(z1, Wc2bd, bc2bd):
    return pl.pallas_call(
        _t1_body,
        grid=(E // 4 // BR,),
        in_specs=[pl.BlockSpec((BR, 128), lambda i: (i, 0)),
                  pl.BlockSpec((128, 64), lambda i: (0, 0)),
                  pl.BlockSpec((1, 64), lambda i: (0, 0))],
        out_specs=pl.BlockSpec((BR, 128), lambda i: (i, 0)),
        out_shape=jax.ShapeDtypeStruct((E // 4, 128), jnp.float32),
    )(z1, Wc2bd, bc2bd)


def _combine_body(p_ref, wa_ref, ba_ref, wb_ref, ha_ref, hb_ref):
    h = p_ref[0] + p_ref[1]
    ha_ref[...] = jnp.dot(h, wa_ref[...], preferred_element_type=jnp.float32) + ba_ref[...]
    hb_ref[...] = jnp.dot(h, wb_ref[...], preferred_element_type=jnp.float32)


def _combine_stage(partial, Wa, be1, Wb):
    return pl.pallas_call(
        _combine_body,
        grid=(N // BN,),
        in_specs=[pl.BlockSpec((2, BN, 16), lambda i: (0, i, 0)),
                  pl.BlockSpec((16, F), lambda i: (0, 0)),
                  pl.BlockSpec((1, F), lambda i: (0, 0)),
                  pl.BlockSpec((16, F), lambda i: (0, 0))],
        out_specs=[pl.BlockSpec((BN, F), lambda i: (i, 0)),
                   pl.BlockSpec((BN, F), lambda i: (i, 0))],
        out_shape=[jax.ShapeDtypeStruct((N, F), jnp.float32),
                   jax.ShapeDtypeStruct((N, F), jnp.float32)],
    )(partial, Wa, be1, Wb)


def _t2_body(z_ref, w2_ref, b2_ref, w3_ref, b3_ref, p1_ref, p2_ref, p3_ref,
             s_ref, g_ref, o_ref):
    e = _elu(z_ref[...])
    e = _elu(jnp.dot(e, w2_ref[...], preferred_element_type=jnp.float32) + b2_ref[...])
    l = jnp.dot(e, w3_ref[...], preferred_element_type=jnp.float32) + b3_ref[...]
    l1 = jnp.dot(l, p1_ref[...], preferred_element_type=jnp.float32)
    l2 = jnp.dot(l, p2_ref[...], preferred_element_type=jnp.float32)
    l3 = jnp.dot(l, p3_ref[...], preferred_element_type=jnp.float32)
    mx = jnp.maximum(jnp.maximum(l, l1), jnp.maximum(l2, l3))
    bmx = jnp.dot(mx, s_ref[...], preferred_element_type=jnp.float32)
    sh = l - bmx
    ssum = jnp.dot(jnp.exp(sh), g_ref[...], preferred_element_type=jnp.float32)
    o_ref[...] = jnp.concatenate(
        [sh - jnp.log(ssum), jnp.zeros((BR, 112), jnp.float32)], axis=1)


def _t2_stage(z2, We2bd, be2bd, We3bd, be3bd, P1, P2, P3, S16, G16):
    full = lambda shape: pl.BlockSpec(shape, lambda i: (0, 0))
    return pl.pallas_call(
        _t2_body,
        grid=(E // 4 // BR,),
        in_specs=[pl.BlockSpec((BR, 128), lambda i: (i, 0)),
                  full((128, 128)), full((1, 128)), full((128, 16)),
                  full((1, 16)), full((16, 16)), full((16, 16)),
                  full((16, 16)), full((16, 16)), full((16, 16))],
        out_specs=pl.BlockSpec((BR, 128), lambda i: (i, 0)),
        out_shape=jax.ShapeDtypeStruct((E // 4, 128), jnp.float32),
    )(z2, We2bd, be2bd, We3bd, be3bd, P1, P2, P3, S16, G16)


@functools.partial(
    pl.kernel,
    out_type=jax.ShapeDtypeStruct((NCH, CB, F), jnp.float32),
    mesh=plsc.VectorSubcoreMesh(core_axis_name="c", subcore_axis_name="s"),
    compiler_params=pltpu.CompilerParams(use_tc_tiling_on_sc=False),
    scratch_types=[
        pltpu.VMEM((8, IB), jnp.int32),
        pltpu.VMEM((8, IB), jnp.int32),
        pltpu.VMEM((8, IB), jnp.int32),
        pltpu.VMEM((CB, F), jnp.float32),
        pltpu.VMEM((CB, F), jnp.float32),
        pltpu.VMEM((CB, F), jnp.float32),
        pltpu.SemaphoreType.DMA,
        pltpu.SemaphoreType.DMA,
        pltpu.SemaphoreType.DMA,
        pltpu.SemaphoreType.DMA,
        pltpu.SemaphoreType.DMA,
    ],
)
def _gather_add(ta_ref, tb_ref, ia_ref, ib_ref, out_ref,
                ia0_v, ia1_v, ib_v, ba0_v, ba1_v, bb_v,
                sa0, sa1, sb, so0, so1):
    wid = lax.axis_index("s") * NC + lax.axis_index("c")
    ia = (ia0_v, ia1_v)
    ba = (ba0_v, ba1_v)
    sa = (sa0, sa1)
    so = (so0, so1)

    def fire_a(c, p):
        gch = wid * NCHUNK + c
        rbase = pl.multiple_of(8 * gch, 8)
        pltpu.sync_copy(ia_ref.at[pl.ds(rbase, 8)], ia[p])
        for b in range(8):
            pltpu.async_copy(ta_ref.at[ia[p].at[b]],
                             ba[p].at[pl.ds(IB * b, IB)], sa[p])

    def fire_b(c):
        gch = wid * NCHUNK + c
        rbase = pl.multiple_of(8 * gch, 8)
        pltpu.sync_copy(ib_ref.at[pl.ds(rbase, 8)], ib_v)
        for b in range(8):
            pltpu.async_copy(tb_ref.at[ib_v.at[b]],
                             bb_v.at[pl.ds(IB * b, IB)], sb)

    def process(c, p):
        gch = wid * NCHUNK + c
        pltpu.make_async_copy(out_ref.at[gch], ba[p], sa[p]).wait()
        pltpu.make_async_copy(out_ref.at[gch], bb_v, sb).wait()

        @pl.when(c + 1 < NCHUNK)
        def _():
            @pl.when(c > 0)
            def _():
                pltpu.make_async_copy(out_ref.at[gch], ba[1 - p], so[1 - p]).wait()
            fire_a(c + 1, 1 - p)

        def add_row(r, _):
            for k in range(2):
                ba[p][r, pl.ds(16 * k, 16)] = (
                    ba[p][r, pl.ds(16 * k, 16)] + bb_v[r, pl.ds(16 * k, 16)])
            return 0

        lax.fori_loop(0, CB, add_row, 0)

        @pl.when(c + 1 < NCHUNK)
        def _():
            fire_b(c + 1)

        pltpu.async_copy(ba[p], out_ref.at[gch], so[p])

    fire_a(0, 0)
    fire_b(0)

    def pair(t, _):
        process(2 * t, 0)
        process(2 * t + 1, 1)
        return 0

    lax.fori_loop(0, NCHUNK // 2, pair, 0)
    pltpu.make_async_copy(out_ref.at[0], ba[0], so[0]).wait()
    pltpu.make_async_copy(out_ref.at[0], ba[1], so[1]).wait()


@functools.partial(
    pl.kernel,
    out_type=jax.ShapeDtypeStruct((NC, N, 16), jnp.float32),
    mesh=plsc.VectorSubcoreMesh(core_axis_name="c", subcore_axis_name="s"),
    compiler_params=pltpu.CompilerParams(use_tc_tiling_on_sc=False),
    scratch_types=[
        pltpu.VMEM((8, IB), jnp.int32),
        pltpu.VMEM((PR * 128,), jnp.float32),
        pltpu.VMEM((CB, 16), jnp.float32),
        pltpu.VMEM_SHARED((N, 16), jnp.float32),
    ],
)
def _scatter_add(m_ref, col_ref, out_ref, idx_v, m_v, mc_v, acc_sh):
    c = lax.axis_index("c")
    s = lax.axis_index("s")
    wid = s * NC + c

    def zrow(r, _):
        mc_v[r, :] = jnp.zeros((16,), jnp.float32)
        return 0

    lax.fori_loop(0, CB, zrow, 0)
    for k in range(3):
        pltpu.sync_copy(mc_v, acc_sh.at[pl.ds(s * N_PER_TILE + k * CB, CB)])
    pltpu.sync_copy(mc_v.at[pl.ds(0, IB)],
                    acc_sh.at[pl.ds(s * N_PER_TILE + 3 * CB, IB)])
    plsc.subcore_barrier()

    def chunk(ch, _):
        gch = wid * NCHUNK + ch
        rbase = pl.multiple_of(8 * gch, 8)
        pltpu.sync_copy(col_ref.at[pl.ds(rbase, 8)], idx_v)
        pltpu.sync_copy(m_ref.at[gch], m_v)

        def unpack_row(r, _):
            for j in range(4):
                mc_v[4 * r + j, :] = m_v[pl.ds(128 * r + 16 * j, 16)]
            return 0

        lax.fori_loop(0, PR, unpack_row, 0)
        for b in range(8):
            pltpu.sync_copy(mc_v.at[pl.ds(IB * b, IB)],
                            acc_sh.at[idx_v.at[b]], add=True)
        return 0

    lax.fori_loop(0, NCHUNK, chunk, 0)
    plsc.subcore_barrier()
    for k in range(3):
        pltpu.sync_copy(acc_sh.at[pl.ds(s * N_PER_TILE + k * CB, CB)], mc_v)
        pltpu.sync_copy(mc_v, out_ref.at[c, pl.ds(s * N_PER_TILE + k * CB, CB)])
    pltpu.sync_copy(acc_sh.at[pl.ds(s * N_PER_TILE + 3 * CB, IB)],
                    mc_v.at[pl.ds(0, IB)])
    pltpu.sync_copy(mc_v.at[pl.ds(0, IB)],
                    out_ref.at[c, pl.ds(s * N_PER_TILE + 3 * CB, IB)])


@functools.partial(
    pl.kernel,
    out_type=jax.ShapeDtypeStruct((NCH, CB * 4), jnp.float32),
    mesh=plsc.VectorSubcoreMesh(core_axis_name="c", subcore_axis_name="s"),
    compiler_params=pltpu.CompilerParams(use_tc_tiling_on_sc=False),
    scratch_types=[
        pltpu.VMEM((PR * 128,), jnp.float32),
        pltpu.VMEM((CB * 4,), jnp.float32),
    ],
)
def _fmt_out(lg_ref, out_ref, m_v, ov_v):
    wid = lax.axis_index("s") * NC + lax.axis_index("c")

    def chunk(ch, _):
        gch = wid * NCHUNK + ch
        pltpu.sync_copy(lg_ref.at[gch], m_v)

        def row(r, _):
            ov_v[pl.ds(16 * r, 16)] = m_v[pl.ds(128 * r, 16)]
            return 0

        lax.fori_loop(0, PR, row, 0)
        pltpu.sync_copy(ov_v, out_ref.at[gch])
        return 0

    lax.fori_loop(0, NCHUNK, chunk, 0)


def kernel(x, edge_index, datanorm, W1, b1, W2, b2, W3, b3,
           Wc1, bc1, Wc2, bc2, We1, be1, We2, be2, We3, be3):
    row3 = edge_index[0].reshape(NCH * 8, IB)
    col3 = edge_index[1].reshape(NCH * 8, IB)
    col3p = col3

    A = Wc1[:21] - Wc1[21:]
    Bm = Wc1[21:]
    Apad = jnp.zeros((32, F), jnp.float32).at[:21, :29].set(A)
    Bpad = jnp.zeros((32, F), jnp.float32).at[:21, :29].set(Bm)
    bc1p = jnp.zeros((1, F), jnp.float32).at[0, :29].set(bc1)
    Wc2p = jnp.zeros((F, 16), jnp.float32).at[:29].set(Wc2)

    eye4 = jnp.eye(4, dtype=jnp.float32)
    Wc2bd = jnp.kron(eye4, Wc2p)
    bc2bd = jnp.tile(bc2, 4).reshape(1, 64)
    We2bd = jnp.kron(eye4, We2)
    be2bd = jnp.tile(be2, 4).reshape(1, 128)
    We3bd = jnp.kron(eye4, We3)
    be3bd = jnp.tile(be3, 4).reshape(1, 16)
    P1 = jnp.eye(16, k=-1, dtype=jnp.float32)
    P2 = jnp.eye(16, k=-2, dtype=jnp.float32)
    P3 = jnp.eye(16, k=-3, dtype=jnp.float32)
    spread = jnp.zeros((4, 4), jnp.float32).at[0].set(1.0)
    S16 = jnp.kron(eye4, spread)
    G16 = jnp.kron(eye4, jnp.ones((4, 4), jnp.float32))

    XA, XB = _node_stage(x, datanorm.reshape(1, 5), W1, b1.reshape(1, 32),
                         W2, b2.reshape(1, 32), W3, b3.reshape(1, 16),
                         Apad, bc1p, Bpad)
    z1 = _gather_add(XA, XB, col3, row3).reshape(E // 4, 128)
    m = _t1_stage(z1, Wc2bd, bc2bd).reshape(NCH, PR * 128)
    partial = _scatter_add(m, col3p)
    HA, HB = _combine_stage(partial, We1[:16], be1.reshape(1, F), We1[16:])
    z2 = _gather_add(HA, HB, row3, col3).reshape(E // 4, 128)
    out = _t2_stage(z2, We2bd, be2bd, We3bd, be3bd, P1, P2, P3, S16, G16)
    return _fmt_out(out.reshape(NCH, PR * 128)).reshape(E, 4)


# speedup vs baseline: 6.4965x; 1.0679x over previous
import functools

import jax
import jax.numpy as jnp
from jax import lax
from jax.experimental import pallas as pl
from jax.experimental.pallas import tpu as pltpu
from jax.experimental.pallas import tpu_sc as plsc

N = 50000
E = 1600000
F = 32
NC = 2
NS = 16
NW = NC * NS
E_PER_W = E // NW
CB = 1000
NCHUNK = E_PER_W // CB
NCH = E // CB
PR = CB // 4
IB = 125
N_PER_TILE = N // NS

BN = 2000
BR = 2000


def _elu(v):
    return jnp.where(v > 0, v, jnp.exp(v) - 1.0)


def _node_body(x_ref, dn_ref, w1_ref, b1_ref, w2_ref, b2_ref, w3_ref, b3_ref,
               wa_ref, ba_ref, wb_ref, xa_ref, xb_ref):
    xn = x_ref[...] * dn_ref[...]
    h = jnp.tanh(jnp.dot(xn, w1_ref[...], preferred_element_type=jnp.float32) + b1_ref[...])
    h = jnp.tanh(jnp.dot(h, w2_ref[...], preferred_element_type=jnp.float32) + b2_ref[...])
    hh = jnp.tanh(jnp.dot(h, w3_ref[...], preferred_element_type=jnp.float32) + b3_ref[...])
    feat = jnp.concatenate([hh, xn, jnp.zeros((BN, 11), jnp.float32)], axis=1)
    xa_ref[...] = jnp.dot(feat, wa_ref[...], preferred_element_type=jnp.float32) + ba_ref[...]
    xb_ref[...] = jnp.dot(feat, wb_ref[...], preferred_element_type=jnp.float32)


def _node_stage(x, dn, W1, b1, W2, b2, W3, b3, Apad, bc1p, Bpad):
    full = lambda shape: pl.BlockSpec(shape, lambda i: (0, 0))
    return pl.pallas_call(
        _node_body,
        grid=(N // BN,),
        in_specs=[
            pl.BlockSpec((BN, 5), lambda i: (i, 0)),
            full((1, 5)), full((5, 32)), full((1, 32)), full((32, 32)),
            full((1, 32)), full((32, 16)), full((1, 16)),
            full((32, F)), full((1, F)), full((32, F)),
        ],
        out_specs=[pl.BlockSpec((BN, F), lambda i: (i, 0)),
                   pl.BlockSpec((BN, F), lambda i: (i, 0))],
        out_shape=[jax.ShapeDtypeStruct((N, F), jnp.float32),
                   jax.ShapeDtypeStruct((N, F), jnp.float32)],
    )(x, dn, W1, b1, W2, b2, W3, b3, Apad, bc1p, Bpad)


def _t1_body(z_ref, wbd_ref, bbd_ref, m_ref):
    p = _elu(z_ref[...])
    mm = _elu(jnp.dot(p, wbd_ref[...], preferred_element_type=jnp.float32) + bbd_ref[...])
    m_ref[...] = jnp.concatenate([mm, jnp.zeros((BR, 64), jnp.float32)], axis=1)


def _t1_stage(z1, Wc2bd, bc2bd):
    return pl.pallas_call(
        _t1_body,
        grid=(E // 4 // BR,),
        in_specs=[pl.BlockSpec((BR, 128), lambda i: (i, 0)),
                  pl.BlockSpec((128, 64), lambda i: (0, 0)),
                  pl.BlockSpec((1, 64), lambda i: (0, 0))],
        out_specs=pl.BlockSpec((BR, 128), lambda i: (i, 0)),
        out_shape=jax.ShapeDtypeStruct((E // 4, 128), jnp.float32),
    )(z1, Wc2bd, bc2bd)


def _combine_body(p_ref, wa_ref, ba_ref, wb_ref, ha_ref, hb_ref):
    h = p_ref[0] + p_ref[1]
    ha_ref[...] = jnp.dot(h, wa_ref[...], preferred_element_type=jnp.float32) + ba_ref[...]
    hb_ref[...] = jnp.dot(h, wb_ref[...], preferred_element_type=jnp.float32)


def _combine_stage(partial, Wa, be1, Wb):
    return pl.pallas_call(
        _combine_body,
        grid=(N // BN,),
        in_specs=[pl.BlockSpec((2, BN, 16), lambda i: (0, i, 0)),
                  pl.BlockSpec((16, F), lambda i: (0, 0)),
                  pl.BlockSpec((1, F), lambda i: (0, 0)),
                  pl.BlockSpec((16, F), lambda i: (0, 0))],
        out_specs=[pl.BlockSpec((BN, F), lambda i: (i, 0)),
                   pl.BlockSpec((BN, F), lambda i: (i, 0))],
        out_shape=[jax.ShapeDtypeStruct((N, F), jnp.float32),
                   jax.ShapeDtypeStruct((N, F), jnp.float32)],
    )(partial, Wa, be1, Wb)


def _t2_body(z_ref, w2_ref, b2_ref, w3_ref, b3_ref, p1_ref, p2_ref, p3_ref,
             s_ref, g_ref, o_ref):
    e = _elu(z_ref[...])
    e = _elu(jnp.dot(e, w2_ref[...], preferred_element_type=jnp.float32) + b2_ref[...])
    l = jnp.dot(e, w3_ref[...], preferred_element_type=jnp.float32) + b3_ref[...]
    l1 = jnp.dot(l, p1_ref[...], preferred_element_type=jnp.float32)
    l2 = jnp.dot(l, p2_ref[...], preferred_element_type=jnp.float32)
    l3 = jnp.dot(l, p3_ref[...], preferred_element_type=jnp.float32)
    mx = jnp.maximum(jnp.maximum(l, l1), jnp.maximum(l2, l3))
    bmx = jnp.dot(mx, s_ref[...], preferred_element_type=jnp.float32)
    sh = l - bmx
    ssum = jnp.dot(jnp.exp(sh), g_ref[...], preferred_element_type=jnp.float32)
    o_ref[...] = jnp.concatenate(
        [sh - jnp.log(ssum), jnp.zeros((BR, 112), jnp.float32)], axis=1)


def _t2_stage(z2, We2bd, be2bd, We3bd, be3bd, P1, P2, P3, S16, G16):
    full = lambda shape: pl.BlockSpec(shape, lambda i: (0, 0))
    return pl.pallas_call(
        _t2_body,
        grid=(E // 4 // BR,),
        in_specs=[pl.BlockSpec((BR, 128), lambda i: (i, 0)),
                  full((128, 128)), full((1, 128)), full((128, 16)),
                  full((1, 16)), full((16, 16)), full((16, 16)),
                  full((16, 16)), full((16, 16)), full((16, 16))],
        out_specs=pl.BlockSpec((BR, 128), lambda i: (i, 0)),
        out_shape=jax.ShapeDtypeStruct((E // 4, 128), jnp.float32),
    )(z2, We2bd, be2bd, We3bd, be3bd, P1, P2, P3, S16, G16)


@functools.partial(
    pl.kernel,
    out_type=jax.ShapeDtypeStruct((NCH, CB, F), jnp.float32),
    mesh=plsc.VectorSubcoreMesh(core_axis_name="c", subcore_axis_name="s"),
    compiler_params=pltpu.CompilerParams(use_tc_tiling_on_sc=False),
    scratch_types=[
        pltpu.VMEM((8, IB), jnp.int32),
        pltpu.VMEM((8, IB), jnp.int32),
        pltpu.VMEM((8, IB), jnp.int32),
        pltpu.VMEM((CB, F), jnp.float32),
        pltpu.VMEM((CB, F), jnp.float32),
        pltpu.VMEM((CB, F), jnp.float32),
        pltpu.SemaphoreType.DMA,
        pltpu.SemaphoreType.DMA,
        pltpu.SemaphoreType.DMA,
        pltpu.SemaphoreType.DMA,
        pltpu.SemaphoreType.DMA,
    ],
)
def _gather_add(ta_ref, tb_ref, ia_ref, ib_ref, out_ref,
                ia0_v, ia1_v, ib_v, ba0_v, ba1_v, bb_v,
                sa0, sa1, sb, so0, so1):
    wid = lax.axis_index("s") * NC + lax.axis_index("c")
    ia = (ia0_v, ia1_v)
    ba = (ba0_v, ba1_v)
    sa = (sa0, sa1)
    so = (so0, so1)

    def fire_a(c, p):
        gch = wid * NCHUNK + c
        rbase = pl.multiple_of(8 * gch, 8)
        pltpu.sync_copy(ia_ref.at[pl.ds(rbase, 8)], ia[p])
        for b in range(8):
            pltpu.async_copy(ta_ref.at[ia[p].at[b]],
                             ba[p].at[pl.ds(IB * b, IB)], sa[p])

    def fire_b(c):
        gch = wid * NCHUNK + c
        rbase = pl.multiple_of(8 * gch, 8)
        pltpu.sync_copy(ib_ref.at[pl.ds(rbase, 8)], ib_v)
        for b in range(8):
            pltpu.async_copy(tb_ref.at[ib_v.at[b]],
                             bb_v.at[pl.ds(IB * b, IB)], sb)

    def process(c, p):
        gch = wid * NCHUNK + c
        pltpu.make_async_copy(out_ref.at[gch], ba[p], sa[p]).wait()
        pltpu.make_async_copy(out_ref.at[gch], bb_v, sb).wait()

        @pl.when(c + 1 < NCHUNK)
        def _():
            @pl.when(c > 0)
            def _():
                pltpu.make_async_copy(out_ref.at[gch], ba[1 - p], so[1 - p]).wait()
            fire_a(c + 1, 1 - p)

        def add_row(r, _):
            for u in range(4):
                for k in range(2):
                    ba[p][4 * r + u, pl.ds(16 * k, 16)] = (
                        ba[p][4 * r + u, pl.ds(16 * k, 16)]
                        + bb_v[4 * r + u, pl.ds(16 * k, 16)])
            return 0

        lax.fori_loop(0, CB // 4, add_row, 0)

        @pl.when(c + 1 < NCHUNK)
        def _():
            fire_b(c + 1)

        pltpu.async_copy(ba[p], out_ref.at[gch], so[p])

    fire_a(0, 0)
    fire_b(0)

    def pair(t, _):
        process(2 * t, 0)
        process(2 * t + 1, 1)
        return 0

    lax.fori_loop(0, NCHUNK // 2, pair, 0)
    pltpu.make_async_copy(out_ref.at[0], ba[0], so[0]).wait()
    pltpu.make_async_copy(out_ref.at[0], ba[1], so[1]).wait()


@functools.partial(
    pl.kernel,
    out_type=jax.ShapeDtypeStruct((NC, N, 16), jnp.float32),
    mesh=plsc.VectorSubcoreMesh(core_axis_name="c", subcore_axis_name="s"),
    compiler_params=pltpu.CompilerParams(use_tc_tiling_on_sc=False),
    scratch_types=[
        pltpu.VMEM((8, IB), jnp.int32),
        pltpu.VMEM((PR * 128,), jnp.float32),
        pltpu.VMEM((CB, 16), jnp.float32),
        pltpu.VMEM_SHARED((N, 16), jnp.float32),
    ],
)
def _scatter_add(m_ref, col_ref, out_ref, idx_v, m_v, mc_v, acc_sh):
    c = lax.axis_index("c")
    s = lax.axis_index("s")
    wid = s * NC + c

    def zrow(r, _):
        mc_v[r, :] = jnp.zeros((16,), jnp.float32)
        return 0

    lax.fori_loop(0, CB, zrow, 0)
    for k in range(3):
        pltpu.sync_copy(mc_v, acc_sh.at[pl.ds(s * N_PER_TILE + k * CB, CB)])
    pltpu.sync_copy(mc_v.at[pl.ds(0, IB)],
                    acc_sh.at[pl.ds(s * N_PER_TILE + 3 * CB, IB)])
    plsc.subcore_barrier()

    def chunk(ch, _):
        gch = wid * NCHUNK + ch
        rbase = pl.multiple_of(8 * gch, 8)
        pltpu.sync_copy(col_ref.at[pl.ds(rbase, 8)], idx_v)
        pltpu.sync_copy(m_ref.at[gch], m_v)

        def unpack_row(r, _):
            for j in range(4):
                mc_v[4 * r + j, :] = m_v[pl.ds(128 * r + 16 * j, 16)]
            return 0

        lax.fori_loop(0, PR, unpack_row, 0)
        for b in range(8):
            pltpu.sync_copy(mc_v.at[pl.ds(IB * b, IB)],
                            acc_sh.at[idx_v.at[b]], add=True)
        return 0

    lax.fori_loop(0, NCHUNK, chunk, 0)
    plsc.subcore_barrier()
    for k in range(3):
        pltpu.sync_copy(acc_sh.at[pl.ds(s * N_PER_TILE + k * CB, CB)], mc_v)
        pltpu.sync_copy(mc_v, out_ref.at[c, pl.ds(s * N_PER_TILE + k * CB, CB)])
    pltpu.sync_copy(acc_sh.at[pl.ds(s * N_PER_TILE + 3 * CB, IB)],
                    mc_v.at[pl.ds(0, IB)])
    pltpu.sync_copy(mc_v.at[pl.ds(0, IB)],
                    out_ref.at[c, pl.ds(s * N_PER_TILE + 3 * CB, IB)])


@functools.partial(
    pl.kernel,
    out_type=jax.ShapeDtypeStruct((NCH, CB * 4), jnp.float32),
    mesh=plsc.VectorSubcoreMesh(core_axis_name="c", subcore_axis_name="s"),
    compiler_params=pltpu.CompilerParams(use_tc_tiling_on_sc=False),
    scratch_types=[
        pltpu.VMEM((PR * 128,), jnp.float32),
        pltpu.VMEM((CB * 4,), jnp.float32),
    ],
)
def _fmt_out(lg_ref, out_ref, m_v, ov_v):
    wid = lax.axis_index("s") * NC + lax.axis_index("c")

    def chunk(ch, _):
        gch = wid * NCHUNK + ch
        pltpu.sync_copy(lg_ref.at[gch], m_v)

        def row(r, _):
            ov_v[pl.ds(16 * r, 16)] = m_v[pl.ds(128 * r, 16)]
            return 0

        lax.fori_loop(0, PR, row, 0)
        pltpu.sync_copy(ov_v, out_ref.at[gch])
        return 0

    lax.fori_loop(0, NCHUNK, chunk, 0)


def kernel(x, edge_index, datanorm, W1, b1, W2, b2, W3, b3,
           Wc1, bc1, Wc2, bc2, We1, be1, We2, be2, We3, be3):
    row3 = edge_index[0].reshape(NCH * 8, IB)
    col3 = edge_index[1].reshape(NCH * 8, IB)
    col3p = col3

    A = Wc1[:21] - Wc1[21:]
    Bm = Wc1[21:]
    Apad = jnp.zeros((32, F), jnp.float32).at[:21, :29].set(A)
    Bpad = jnp.zeros((32, F), jnp.float32).at[:21, :29].set(Bm)
    bc1p = jnp.zeros((1, F), jnp.float32).at[0, :29].set(bc1)
    Wc2p = jnp.zeros((F, 16), jnp.float32).at[:29].set(Wc2)

    eye4 = jnp.eye(4, dtype=jnp.float32)
    Wc2bd = jnp.kron(eye4, Wc2p)
    bc2bd = jnp.tile(bc2, 4).reshape(1, 64)
    We2bd = jnp.kron(eye4, We2)
    be2bd = jnp.tile(be2, 4).reshape(1, 128)
    We3bd = jnp.kron(eye4, We3)
    be3bd = jnp.tile(be3, 4).reshape(1, 16)
    P1 = jnp.eye(16, k=-1, dtype=jnp.float32)
    P2 = jnp.eye(16, k=-2, dtype=jnp.float32)
    P3 = jnp.eye(16, k=-3, dtype=jnp.float32)
    spread = jnp.zeros((4, 4), jnp.float32).at[0].set(1.0)
    S16 = jnp.kron(eye4, spread)
    G16 = jnp.kron(eye4, jnp.ones((4, 4), jnp.float32))

    XA, XB = _node_stage(x, datanorm.reshape(1, 5), W1, b1.reshape(1, 32),
                         W2, b2.reshape(1, 32), W3, b3.reshape(1, 16),
                         Apad, bc1p, Bpad)
    z1 = _gather_add(XA, XB, col3, row3).reshape(E // 4, 128)
    m = _t1_stage(z1, Wc2bd, bc2bd).reshape(NCH, PR * 128)
    partial = _scatter_add(m, col3p)
    HA, HB = _combine_stage(partial, We1[:16], be1.reshape(1, F), We1[16:])
    z2 = _gather_add(HA, HB, row3, col3).reshape(E // 4, 128)
    out = _t2_stage(z2, We2bd, be2bd, We3bd, be3bd, P1, P2, P3, S16, G16)
    return _fmt_out(out.reshape(NCH, PR * 128)).reshape(E, 4)


# speedup vs baseline: 6.5533x; 1.0087x over previous
import functools

import jax
import jax.numpy as jnp
from jax import lax
from jax.experimental import pallas as pl
from jax.experimental.pallas import tpu as pltpu
from jax.experimental.pallas import tpu_sc as plsc

N = 50000
E = 1600000
F = 32
NC = 2
NS = 16
NW = NC * NS
E_PER_W = E // NW
CB = 1000
NCHUNK = E_PER_W // CB
NCH = E // CB
PR = CB // 4
IB = 125
N_PER_TILE = N // NS

BN = 2000
BR = 2000


def _elu(v):
    return jnp.where(v > 0, v, jnp.exp(v) - 1.0)


def _node_body(x_ref, dn_ref, w1_ref, b1_ref, w2_ref, b2_ref, w3_ref, b3_ref,
               wa_ref, ba_ref, wb_ref, xa_ref, xb_ref):
    xn = x_ref[...] * dn_ref[...]
    h = jnp.tanh(jnp.dot(xn, w1_ref[...], preferred_element_type=jnp.float32) + b1_ref[...])
    h = jnp.tanh(jnp.dot(h, w2_ref[...], preferred_element_type=jnp.float32) + b2_ref[...])
    hh = jnp.tanh(jnp.dot(h, w3_ref[...], preferred_element_type=jnp.float32) + b3_ref[...])
    feat = jnp.concatenate([hh, xn, jnp.zeros((BN, 11), jnp.float32)], axis=1)
    xa_ref[...] = jnp.dot(feat, wa_ref[...], preferred_element_type=jnp.float32) + ba_ref[...]
    xb_ref[...] = jnp.dot(feat, wb_ref[...], preferred_element_type=jnp.float32)


def _node_stage(x, dn, W1, b1, W2, b2, W3, b3, Apad, bc1p, Bpad):
    full = lambda shape: pl.BlockSpec(shape, lambda i: (0, 0))
    return pl.pallas_call(
        _node_body,
        grid=(N // BN,),
        in_specs=[
            pl.BlockSpec((BN, 5), lambda i: (i, 0)),
            full((1, 5)), full((5, 32)), full((1, 32)), full((32, 32)),
            full((1, 32)), full((32, 16)), full((1, 16)),
            full((32, F)), full((1, F)), full((32, F)),
        ],
        out_specs=[pl.BlockSpec((BN, F), lambda i: (i, 0)),
                   pl.BlockSpec((BN, F), lambda i: (i, 0))],
        out_shape=[jax.ShapeDtypeStruct((N, F), jnp.float32),
                   jax.ShapeDtypeStruct((N, F), jnp.float32)],
    )(x, dn, W1, b1, W2, b2, W3, b3, Apad, bc1p, Bpad)


def _t1_body(z_ref, wbd_ref, bbd_ref, m_ref):
    p = _elu(z_ref[...])
    mm = _elu(jnp.dot(p, wbd_ref[...], preferred_element_type=jnp.float32) + bbd_ref[...])
    m_ref[...] = jnp.concatenate([mm, jnp.zeros((BR, 64), jnp.float32)], axis=1)


def _t1_stage(z1, Wc2bd, bc2bd):
    return pl.pallas_call(
        _t1_body,
        grid=(E // 4 // BR,),
        in_specs=[pl.BlockSpec((BR, 128), lambda i: (i, 0)),
                  pl.BlockSpec((128, 64), lambda i: (0, 0)),
                  pl.BlockSpec((1, 64), lambda i: (0, 0))],
        out_specs=pl.BlockSpec((BR, 128), lambda i: (i, 0)),
        out_shape=jax.ShapeDtypeStruct((E // 4, 128), jnp.float32),
    )(z1, Wc2bd, bc2bd)


def _combine_body(p_ref, wa_ref, ba_ref, wb_ref, ha_ref, hb_ref):
    h = p_ref[0] + p_ref[1]
    ha_ref[...] = jnp.dot(h, wa_ref[...], preferred_element_type=jnp.float32) + ba_ref[...]
    hb_ref[...] = jnp.dot(h, wb_ref[...], preferred_element_type=jnp.float32)


def _combine_stage(partial, Wa, be1, Wb):
    return pl.pallas_call(
        _combine_body,
        grid=(N // BN,),
        in_specs=[pl.BlockSpec((2, BN, 16), lambda i: (0, i, 0)),
                  pl.BlockSpec((16, F), lambda i: (0, 0)),
                  pl.BlockSpec((1, F), lambda i: (0, 0)),
                  pl.BlockSpec((16, F), lambda i: (0, 0))],
        out_specs=[pl.BlockSpec((BN, F), lambda i: (i, 0)),
                   pl.BlockSpec((BN, F), lambda i: (i, 0))],
        out_shape=[jax.ShapeDtypeStruct((N, F), jnp.float32),
                   jax.ShapeDtypeStruct((N, F), jnp.float32)],
    )(partial, Wa, be1, Wb)


def _t2_body(z_ref, w2_ref, b2_ref, w3_ref, b3_ref, p1_ref, p2_ref, p3_ref,
             s_ref, g_ref, o_ref):
    e = _elu(z_ref[...])
    e = _elu(jnp.dot(e, w2_ref[...], preferred_element_type=jnp.float32) + b2_ref[...])
    l = jnp.dot(e, w3_ref[...], preferred_element_type=jnp.float32) + b3_ref[...]
    l1 = jnp.dot(l, p1_ref[...], preferred_element_type=jnp.float32)
    l2 = jnp.dot(l, p2_ref[...], preferred_element_type=jnp.float32)
    l3 = jnp.dot(l, p3_ref[...], preferred_element_type=jnp.float32)
    mx = jnp.maximum(jnp.maximum(l, l1), jnp.maximum(l2, l3))
    bmx = jnp.dot(mx, s_ref[...], preferred_element_type=jnp.float32)
    sh = l - bmx
    ssum = jnp.dot(jnp.exp(sh), g_ref[...], preferred_element_type=jnp.float32)
    o_ref[...] = jnp.concatenate(
        [sh - jnp.log(ssum), jnp.zeros((BR, 112), jnp.float32)], axis=1)


def _t2_stage(z2, We2bd, be2bd, We3bd, be3bd, P1, P2, P3, S16, G16):
    full = lambda shape: pl.BlockSpec(shape, lambda i: (0, 0))
    return pl.pallas_call(
        _t2_body,
        grid=(E // 4 // BR,),
        in_specs=[pl.BlockSpec((BR, 128), lambda i: (i, 0)),
                  full((128, 128)), full((1, 128)), full((128, 16)),
                  full((1, 16)), full((16, 16)), full((16, 16)),
                  full((16, 16)), full((16, 16)), full((16, 16))],
        out_specs=pl.BlockSpec((BR, 128), lambda i: (i, 0)),
        out_shape=jax.ShapeDtypeStruct((E // 4, 128), jnp.float32),
    )(z2, We2bd, be2bd, We3bd, be3bd, P1, P2, P3, S16, G16)


@functools.partial(
    pl.kernel,
    out_type=jax.ShapeDtypeStruct((NCH, CB, F), jnp.float32),
    mesh=plsc.VectorSubcoreMesh(core_axis_name="c", subcore_axis_name="s"),
    compiler_params=pltpu.CompilerParams(use_tc_tiling_on_sc=False),
    scratch_types=[
        pltpu.VMEM((8, IB), jnp.int32),
        pltpu.VMEM((8, IB), jnp.int32),
        pltpu.VMEM((8, IB), jnp.int32),
        pltpu.VMEM((CB, F), jnp.float32),
        pltpu.VMEM((CB, F), jnp.float32),
        pltpu.VMEM((CB, F), jnp.float32),
        pltpu.SemaphoreType.DMA,
        pltpu.SemaphoreType.DMA,
        pltpu.SemaphoreType.DMA,
        pltpu.SemaphoreType.DMA,
        pltpu.SemaphoreType.DMA,
    ],
)
def _gather_add(ta_ref, tb_ref, ia_ref, ib_ref, out_ref,
                ia0_v, ia1_v, ib_v, ba0_v, ba1_v, bb_v,
                sa0, sa1, sb, so0, so1):
    wid = lax.axis_index("s") * NC + lax.axis_index("c")
    ia = (ia0_v, ia1_v)
    ba = (ba0_v, ba1_v)
    sa = (sa0, sa1)
    so = (so0, so1)

    def fire_a(c, p):
        gch = wid * NCHUNK + c
        rbase = pl.multiple_of(8 * gch, 8)
        pltpu.sync_copy(ia_ref.at[pl.ds(rbase, 8)], ia[p])
        for b in range(8):
            pltpu.async_copy(ta_ref.at[ia[p].at[b]],
                             ba[p].at[pl.ds(IB * b, IB)], sa[p])

    def fire_b(c):
        gch = wid * NCHUNK + c
        rbase = pl.multiple_of(8 * gch, 8)
        pltpu.sync_copy(ib_ref.at[pl.ds(rbase, 8)], ib_v)
        for b in range(8):
            pltpu.async_copy(tb_ref.at[ib_v.at[b]],
                             bb_v.at[pl.ds(IB * b, IB)], sb)

    def process(c, p):
        gch = wid * NCHUNK + c
        pltpu.make_async_copy(out_ref.at[gch], ba[p], sa[p]).wait()
        pltpu.make_async_copy(out_ref.at[gch], bb_v, sb).wait()

        @pl.when(c + 1 < NCHUNK)
        def _():
            @pl.when(c > 0)
            def _():
                pltpu.make_async_copy(out_ref.at[gch], ba[1 - p], so[1 - p]).wait()
            fire_a(c + 1, 1 - p)

        def add_row(r, _):
            for u in range(4):
                for k in range(2):
                    ba[p][4 * r + u, pl.ds(16 * k, 16)] = (
                        ba[p][4 * r + u, pl.ds(16 * k, 16)]
                        + bb_v[4 * r + u, pl.ds(16 * k, 16)])
            return 0

        lax.fori_loop(0, CB // 4, add_row, 0)

        @pl.when(c + 1 < NCHUNK)
        def _():
            fire_b(c + 1)

        pltpu.async_copy(ba[p], out_ref.at[gch], so[p])

    fire_a(0, 0)
    fire_b(0)

    def pair(t, _):
        process(2 * t, 0)
        process(2 * t + 1, 1)
        return 0

    lax.fori_loop(0, NCHUNK // 2, pair, 0)
    pltpu.make_async_copy(out_ref.at[0], ba[0], so[0]).wait()
    pltpu.make_async_copy(out_ref.at[0], ba[1], so[1]).wait()


@functools.partial(
    pl.kernel,
    out_type=jax.ShapeDtypeStruct((NC, N, 16), jnp.float32),
    mesh=plsc.VectorSubcoreMesh(core_axis_name="c", subcore_axis_name="s"),
    compiler_params=pltpu.CompilerParams(use_tc_tiling_on_sc=False),
    scratch_types=[
        pltpu.VMEM((8, IB), jnp.int32),
        pltpu.VMEM((PR * 128,), jnp.float32),
        pltpu.VMEM((CB, 16), jnp.float32),
        pltpu.VMEM_SHARED((N, 16), jnp.float32),
    ],
)
def _scatter_add(m_ref, col_ref, out_ref, idx_v, m_v, mc_v, acc_sh):
    c = lax.axis_index("c")
    s = lax.axis_index("s")
    wid = s * NC + c

    def zrow(r, _):
        for u in range(4):
            mc_v[4 * r + u, :] = jnp.zeros((16,), jnp.float32)
        return 0

    lax.fori_loop(0, CB // 4, zrow, 0)
    for k in range(3):
        pltpu.sync_copy(mc_v, acc_sh.at[pl.ds(s * N_PER_TILE + k * CB, CB)])
    pltpu.sync_copy(mc_v.at[pl.ds(0, IB)],
                    acc_sh.at[pl.ds(s * N_PER_TILE + 3 * CB, IB)])
    plsc.subcore_barrier()

    def chunk(ch, _):
        gch = wid * NCHUNK + ch
        rbase = pl.multiple_of(8 * gch, 8)
        pltpu.sync_copy(col_ref.at[pl.ds(rbase, 8)], idx_v)
        pltpu.sync_copy(m_ref.at[gch], m_v)

        def unpack_row(r, _):
            for u in range(2):
                for j in range(4):
                    mc_v[8 * r + 4 * u + j, :] = (
                        m_v[pl.ds(256 * r + 128 * u + 16 * j, 16)])
            return 0

        lax.fori_loop(0, PR // 2, unpack_row, 0)
        for b in range(8):
            pltpu.sync_copy(mc_v.at[pl.ds(IB * b, IB)],
                            acc_sh.at[idx_v.at[b]], add=True)
        return 0

    lax.fori_loop(0, NCHUNK, chunk, 0)
    plsc.subcore_barrier()
    for k in range(3):
        pltpu.sync_copy(acc_sh.at[pl.ds(s * N_PER_TILE + k * CB, CB)], mc_v)
        pltpu.sync_copy(mc_v, out_ref.at[c, pl.ds(s * N_PER_TILE + k * CB, CB)])
    pltpu.sync_copy(acc_sh.at[pl.ds(s * N_PER_TILE + 3 * CB, IB)],
                    mc_v.at[pl.ds(0, IB)])
    pltpu.sync_copy(mc_v.at[pl.ds(0, IB)],
                    out_ref.at[c, pl.ds(s * N_PER_TILE + 3 * CB, IB)])


@functools.partial(
    pl.kernel,
    out_type=jax.ShapeDtypeStruct((NCH, CB * 4), jnp.float32),
    mesh=plsc.VectorSubcoreMesh(core_axis_name="c", subcore_axis_name="s"),
    compiler_params=pltpu.CompilerParams(use_tc_tiling_on_sc=False),
    scratch_types=[
        pltpu.VMEM((PR * 128,), jnp.float32),
        pltpu.VMEM((CB * 4,), jnp.float32),
    ],
)
def _fmt_out(lg_ref, out_ref, m_v, ov_v):
    wid = lax.axis_index("s") * NC + lax.axis_index("c")

    def chunk(ch, _):
        gch = wid * NCHUNK + ch
        pltpu.sync_copy(lg_ref.at[gch], m_v)

        def row(r, _):
            for u in range(5):
                ov_v[pl.ds(16 * (5 * r + u), 16)] = m_v[pl.ds(128 * (5 * r + u), 16)]
            return 0

        lax.fori_loop(0, PR // 5, row, 0)
        pltpu.sync_copy(ov_v, out_ref.at[gch])
        return 0

    lax.fori_loop(0, NCHUNK, chunk, 0)


def kernel(x, edge_index, datanorm, W1, b1, W2, b2, W3, b3,
           Wc1, bc1, Wc2, bc2, We1, be1, We2, be2, We3, be3):
    row3 = edge_index[0].reshape(NCH * 8, IB)
    col3 = edge_index[1].reshape(NCH * 8, IB)
    col3p = col3

    A = Wc1[:21] - Wc1[21:]
    Bm = Wc1[21:]
    Apad = jnp.zeros((32, F), jnp.float32).at[:21, :29].set(A)
    Bpad = jnp.zeros((32, F), jnp.float32).at[:21, :29].set(Bm)
    bc1p = jnp.zeros((1, F), jnp.float32).at[0, :29].set(bc1)
    Wc2p = jnp.zeros((F, 16), jnp.float32).at[:29].set(Wc2)

    eye4 = jnp.eye(4, dtype=jnp.float32)
    Wc2bd = jnp.kron(eye4, Wc2p)
    bc2bd = jnp.tile(bc2, 4).reshape(1, 64)
    We2bd = jnp.kron(eye4, We2)
    be2bd = jnp.tile(be2, 4).reshape(1, 128)
    We3bd = jnp.kron(eye4, We3)
    be3bd = jnp.tile(be3, 4).reshape(1, 16)
    P1 = jnp.eye(16, k=-1, dtype=jnp.float32)
    P2 = jnp.eye(16, k=-2, dtype=jnp.float32)
    P3 = jnp.eye(16, k=-3, dtype=jnp.float32)
    spread = jnp.zeros((4, 4), jnp.float32).at[0].set(1.0)
    S16 = jnp.kron(eye4, spread)
    G16 = jnp.kron(eye4, jnp.ones((4, 4), jnp.float32))

    XA, XB = _node_stage(x, datanorm.reshape(1, 5), W1, b1.reshape(1, 32),
                         W2, b2.reshape(1, 32), W3, b3.reshape(1, 16),
                         Apad, bc1p, Bpad)
    z1 = _gather_add(XA, XB, col3, row3).reshape(E // 4, 128)
    m = _t1_stage(z1, Wc2bd, bc2bd).reshape(NCH, PR * 128)
    partial = _scatter_add(m, col3p)
    HA, HB = _combine_stage(partial, We1[:16], be1.reshape(1, F), We1[16:])
    z2 = _gather_add(HA, HB, row3, col3).reshape(E // 4, 128)
    out = _t2_stage(z2, We2bd, be2bd, We3bd, be3bd, P1, P2, P3, S16, G16)
    return _fmt_out(out.reshape(NCH, PR * 128)).reshape(E, 4)
